# baseline (speedup 1.0000x reference)
import functools

import jax
import jax.numpy as jnp
from jax import lax
from jax.experimental import pallas as pl
from jax.experimental.pallas import tpu as pltpu
from jax.experimental.pallas import tpu_sc as plsc

B = 30000
S = 16
F = 128
E = 64
L = 16
NC = 2
NS = 16
NW = NC * NS
BPW = 1024
B_PAD = NW * BPW
G = 128
RPG = G // S


def _sc_body(nodes_hbm, neigh_hbm, feat_hbm, self_out, mean_out,
             idxn_v, idxs_v, nbuf, sbuf, mbuf, sem):
  c = lax.axis_index("c")
  s = lax.axis_index("s")
  wid = s * NC + c
  base = wid * BPW

  pltpu.sync_copy(neigh_hbm.at[pl.ds(wid * (BPW * S // G), BPW * S // G)],
                  idxn_v)
  pltpu.sync_copy(nodes_hbm.at[pl.ds(wid * (BPW // G), BPW // G)], idxs_v)

  def self_step(j, carry):
    pltpu.async_copy(feat_hbm.at[idxs_v.at[j]], sbuf, sem).wait()
    pltpu.sync_copy(sbuf, self_out.at[pl.ds(base + j * G, G)])
    return carry

  lax.fori_loop(0, BPW // G, self_step, 0)

  inv_s = jnp.float32(1.0 / S)

  def group_step(g, carry):
    def t_step(t, carry2):
      i = g * S + t
      pltpu.async_copy(feat_hbm.at[idxn_v.at[i]], nbuf, sem).wait()

      def row_step(r, carry3):
        rbase = r * S
        mrow = t * RPG + r
        for cc in range(F // L):
          col = pl.ds(cc * L, L)
          acc = nbuf[rbase, col]
          for sn in range(1, S):
            acc = acc + nbuf[rbase + sn, col]
          mbuf[mrow, col] = acc * inv_s
        return carry3

      lax.fori_loop(0, RPG, row_step, 0)
      return carry2

    lax.fori_loop(0, S, t_step, 0)
    pltpu.sync_copy(mbuf, mean_out.at[pl.ds(base + g * G, G)])
    return carry

  lax.fori_loop(0, BPW // G, group_step, 0)


@jax.jit
def _sc_gather_mean(nodes2d, neigh2d, feat_table):
  mesh = plsc.VectorSubcoreMesh(core_axis_name="c", subcore_axis_name="s",
                                num_cores=NC, num_subcores=NS)
  f32 = jnp.float32
  out_type = (jax.ShapeDtypeStruct((B_PAD, F), f32),
              jax.ShapeDtypeStruct((B_PAD, F), f32))
  scratch = [
      pltpu.VMEM((BPW * S // G, G), jnp.int32),
      pltpu.VMEM((BPW // G, G), jnp.int32),
      pltpu.VMEM((G, F), f32),
      pltpu.VMEM((G, F), f32),
      pltpu.VMEM((G, F), f32),
      pltpu.SemaphoreType.DMA,
  ]
  return pl.kernel(_sc_body, out_type=out_type, mesh=mesh,
                   scratch_types=scratch)(nodes2d, neigh2d, feat_table)


def _tc_body(x1_ref, x2_ref, w1_ref, w2_ref, b_ref, o_ref):
  y = jnp.dot(x1_ref[...], w1_ref[...], preferred_element_type=jnp.float32)
  y = y + jnp.dot(x2_ref[...], w2_ref[...], preferred_element_type=jnp.float32)
  y = y + b_ref[...]
  o_ref[...] = y * jax.nn.sigmoid(y)


@functools.partial(jax.jit, static_argnames=("bt",))
def _tc_linear_swish(x1, x2, w1, w2, b2d, bt=2048):
  nblk = B_PAD // bt
  return pl.pallas_call(
      _tc_body,
      grid=(nblk,),
      in_specs=[
          pl.BlockSpec((bt, F), lambda i: (i, 0)),
          pl.BlockSpec((bt, F), lambda i: (i, 0)),
          pl.BlockSpec((F, E), lambda i: (0, 0)),
          pl.BlockSpec((F, E), lambda i: (0, 0)),
          pl.BlockSpec((1, E), lambda i: (0, 0)),
      ],
      out_specs=pl.BlockSpec((bt, E), lambda i: (i, 0)),
      out_shape=jax.ShapeDtypeStruct((B_PAD, E), jnp.float32),
  )(x1, x2, w1, w2, b2d)


def kernel(nodes, neigh_idx, feat_table, W, b):
  pad = B_PAD - B
  nodes_p = jnp.pad(nodes, (0, pad)).reshape(B_PAD // G, G)
  neigh_p = jnp.pad(neigh_idx, ((0, pad), (0, 0))).reshape(B_PAD * S // G, G)
  self_f, mean_f = _sc_gather_mean(nodes_p, neigh_p, feat_table)
  out = _tc_linear_swish(self_f, mean_f, W[:F], W[F:], b.reshape(1, E))
  return out[:B]
